# baseline (speedup 1.0000x reference)
import functools

import jax
import jax.numpy as jnp
import numpy as np
from jax.experimental import pallas as pl
from jax.experimental.pallas import tpu as pltpu
from jax.sharding import Mesh, PartitionSpec as P

shard_map = jax.shard_map

AS, BT, HS, HV, ES, EV, NH, CB = 16, 5, 128, 16, 64, 8, 8, 3
B, N = 8, 128
DH = HS // NH
NE = N * N

_SELU_A = 1.6732632423543772
_SELU_S = 1.0507009873554805


def _selu(x):
    return _SELU_S * jnp.where(x > 0, x, _SELU_A * (jnp.exp(x) - 1.0))


def _ln_rows(z, s, b):
    mu = jnp.mean(z, axis=-1, keepdims=True)
    d = z - mu
    var = jnp.mean(d * d, axis=-1, keepdims=True)
    return d * jax.lax.rsqrt(var + 1e-5) * s + b


def _dot(a, b):
    return jnp.dot(a, b, preferred_element_type=jnp.float32)


def _fwd_kernel(
    patoms_ref, ratoms_ref, padjt_ref, radjt_ref, maskr_ref, maskc_ref,
    xembW_ref, xembB_ref, eembWT_ref, eembBc_ref,
    Wqkvog_ref, b128_ref, WedgeT_ref, WeT_ref, b64c_ref,
    gsxW_ref, gsxB_ref, gseWT_ref, gseBc_ref,
    tsW_ref, tsB_ref, ggW_ref, ggB_ref, ggWf_ref, ggBf_ref,
    out_ref, eP, eR,
):
    mask_row = maskr_ref[0]

    def layer_step(l, x, e_scr):
        E = e_scr[:, :]
        EB = _dot(WeT_ref[l], E)
        Wq = Wqkvog_ref[l, 0]
        Wk = Wqkvog_ref[l, 1]
        Wv = Wqkvog_ref[l, 2]
        q = _dot(x, Wq)
        k = _dot(x, Wk)
        v = _dot(x, Wv)
        outs = []
        for h in range(NH):
            q_h = q[:, h * DH:(h + 1) * DH]
            k_h = k[:, h * DH:(h + 1) * DH]
            v_h = v[:, h * DH:(h + 1) * DH]
            qk = jax.lax.dot_general(
                q_h, k_h, (((1,), (1,)), ((), ())),
                preferred_element_type=jnp.float32)
            eb_h = EB[h:h + 1, :].reshape(1, N, N)[0]
            logits = qk * (1.0 / np.sqrt(DH)) + eb_h
            logits = jnp.where(mask_row > 0, logits, -1e6)
            m = jnp.max(logits, axis=-1, keepdims=True)
            p = jnp.exp(logits - m)
            s = jnp.sum(p, axis=-1, keepdims=True)
            outs.append(_dot(p / s, v_h))
        out = jnp.concatenate(outs, axis=-1)
        out = _dot(out, Wqkvog_ref[l, 3]) + b128_ref[l, 0]
        g = jax.nn.sigmoid(_dot(x, Wqkvog_ref[l, 4]) + b128_ref[l, 1])
        x = _ln_rows(x + g * out, b128_ref[l, 2], b128_ref[l, 3])
        Hm = _selu(_dot(WedgeT_ref[l], E) + b64c_ref[l, 0])
        Z = E + Hm
        ones_row = jnp.full((1, ES), 1.0 / ES, dtype=jnp.float32)
        mu = _dot(ones_row, Z)
        ex2 = _dot(ones_row, Z * Z)
        var = ex2 - mu * mu
        e_scr[:, :] = ((Z - mu) * jax.lax.rsqrt(var + 1e-5) * b64c_ref[l, 1]
                       + b64c_ref[l, 2])
        return x

    def stack3(layer_base, x, e_scr):
        return jax.lax.fori_loop(
            layer_base, layer_base + CB,
            lambda l, xc: layer_step(l, xc, e_scr), x)

    def encode(enc_i, atoms_ref, adjt_ref, e_scr, layer_base):
        x = _dot(atoms_ref[0], xembW_ref[enc_i]) + xembB_ref[enc_i]
        e_scr[:, :] = _dot(eembWT_ref[enc_i], adjt_ref[0]) + eembBc_ref[enc_i]
        return stack3(layer_base, x, e_scr)

    p_x = encode(0, patoms_ref, padjt_ref, eP, 0)
    r_x = encode(1, ratoms_ref, radjt_ref, eR, CB)

    gx = jax.nn.sigmoid(_dot(r_x, gsxW_ref[0]) + _dot(p_x, gsxW_ref[1])
                        + gsxB_ref[0])
    xs = gx * r_x + (1.0 - gx) * p_x
    Er = eR[:, :]
    Ep = eP[:, :]
    ge = jax.nn.sigmoid(_dot(gseWT_ref[0], Er) + _dot(gseWT_ref[1], Ep)
                        + gseBc_ref[0])
    eR[:, :] = ge * Er + (1.0 - ge) * Ep

    xs = stack3(2 * CB, xs, eR)

    h = xs
    for i in range(3):
        h = _selu(_dot(h, tsW_ref[i]) + tsB_ref[i])

    a = _selu(_dot(h, ggW_ref[0]) + _dot(xs, ggW_ref[1]) + ggB_ref[0])
    for i in range(3):
        a = _selu(_dot(a, ggW_ref[2 + i]) + ggB_ref[1 + i])
    energies = _selu(_dot(a, ggWf_ref[0]) + ggBf_ref[0:1, 0:1])
    energies = energies - (maskc_ref[0] == 0).astype(jnp.float32) * 1e6
    m = jnp.max(energies, axis=0, keepdims=True)
    p = jnp.exp(energies - m)
    att = p / jnp.sum(p, axis=0, keepdims=True)

    b = _selu(_dot(h, ggW_ref[5]) + ggB_ref[4])
    for i in range(3):
        b = _selu(_dot(b, ggW_ref[6 + i]) + ggB_ref[5 + i])
    emb = _selu(_dot(b, ggWf_ref[1]) + ggBf_ref[1:2, 0:1])

    out_ref[0] = jnp.sum(att * emb, axis=0, keepdims=True)


def _pack(params):
    p_enc = params["rencoder"]
    r_enc = params["pencoder"]
    layers = list(p_enc["layers"]) + list(r_enc["layers"]) + list(params["ts_layers"])
    Wqkvog = jnp.stack([
        jnp.stack([lp["Wq"], lp["Wk"], lp["Wv"], lp["Wo"]["W"], lp["gate"]["W"]])
        for lp in layers])
    b128 = jnp.stack([
        jnp.stack([lp["Wo"]["b"], lp["gate"]["b"], lp["ln_s"], lp["ln_b"]])
        for lp in layers])
    WedgeT = jnp.stack([lp["edge"]["W"].T for lp in layers])
    WeT = jnp.stack([lp["We"].T for lp in layers])
    b64c = jnp.stack([
        jnp.stack([lp["edge"]["b"], lp["lne_s"], lp["lne_b"]])
        for lp in layers])[..., None]
    xembW = jnp.stack([p_enc["x_emb"]["W"], r_enc["x_emb"]["W"]])
    xembB = jnp.stack([p_enc["x_emb"]["b"], r_enc["x_emb"]["b"]])
    eembWT = jnp.stack([p_enc["e_emb"]["W"].T, r_enc["e_emb"]["W"].T])
    eembBc = jnp.stack([p_enc["e_emb"]["b"], r_enc["e_emb"]["b"]])[..., None]
    gsx = params["x_gate"]["gs"]
    gsxW = jnp.stack([gsx["W"][:HS], gsx["W"][HS:]])
    gsxB = gsx["b"][None, :]
    gse = params["e_gate"]["gs"]
    gseWT = jnp.stack([gse["W"][:ES].T, gse["W"][ES:].T])
    gseBc = gse["b"][None, :, None]
    tsW = jnp.stack([lp["W"] for lp in params["tsmlp"]])
    tsB = jnp.stack([lp["b"] for lp in params["tsmlp"]])
    att = params["gg"]["att"]
    emb = params["gg"]["emb"]
    ggW = jnp.stack([att[0]["W"][:HS], att[0]["W"][HS:],
                     att[1]["W"], att[2]["W"], att[3]["W"],
                     emb[0]["W"], emb[1]["W"], emb[2]["W"], emb[3]["W"]])
    ggB = jnp.stack([att[0]["b"], att[1]["b"], att[2]["b"], att[3]["b"],
                     emb[0]["b"], emb[1]["b"], emb[2]["b"], emb[3]["b"]])
    ggWf = jnp.stack([att[4]["W"], emb[4]["W"]])
    ggBf = jnp.stack([att[4]["b"], emb[4]["b"]])
    return (xembW, xembB, eembWT, eembBc, Wqkvog, b128, WedgeT, WeT, b64c,
            gsxW, gsxB, gseWT, gseBc, tsW, tsB, ggW, ggB, ggWf, ggBf)


def _full(shape):
    nd = len(shape)
    return pl.BlockSpec(shape, lambda b, _n=nd: (0,) * _n)


def _run(patoms, ratoms, padjs, radjs, masks, packed, interpret, bl):
    padjt = padjs.transpose(0, 3, 1, 2).reshape(bl, BT, NE)
    radjt = radjs.transpose(0, 3, 1, 2).reshape(bl, BT, NE)
    maskr = masks[:, None, :]
    maskc = masks[..., None]
    in_specs = [
        pl.BlockSpec((1, N, AS), lambda b: (b, 0, 0)),
        pl.BlockSpec((1, N, AS), lambda b: (b, 0, 0)),
        pl.BlockSpec((1, BT, NE), lambda b: (b, 0, 0)),
        pl.BlockSpec((1, BT, NE), lambda b: (b, 0, 0)),
        pl.BlockSpec((1, 1, N), lambda b: (b, 0, 0)),
        pl.BlockSpec((1, N, 1), lambda b: (b, 0, 0)),
    ] + [_full(w.shape) for w in packed]
    out = pl.pallas_call(
        _fwd_kernel,
        grid=(bl,),
        in_specs=in_specs,
        out_specs=pl.BlockSpec((1, 1, 1), lambda b: (b, 0, 0)),
        out_shape=jax.ShapeDtypeStruct((bl, 1, 1), jnp.float32),
        scratch_shapes=[pltpu.VMEM((ES, NE), jnp.float32),
                        pltpu.VMEM((ES, NE), jnp.float32)],
        compiler_params=pltpu.CompilerParams(
            dimension_semantics=("parallel",),
            vmem_limit_bytes=100 * 1024 * 1024,
        ),
        interpret=interpret,
    )(patoms, ratoms, padjt, radjt, maskr, maskc, *packed)
    return out.reshape(bl, 1)


@functools.partial(jax.jit, static_argnames=("interpret",))
def _forward(ratoms, patoms, radjs, padjs, masks, params, interpret=False):
    packed = _pack(params)
    devs = jax.devices()
    tpus = [d for d in devs if d.platform == "tpu"]
    devs = tpus if tpus else devs
    ndev = max(k for k in (8, 4, 2, 1) if k <= len(devs) and B % k == 0)
    if ndev == 1:
        return _run(patoms, ratoms, padjs, radjs, masks, packed, interpret, B)
    mesh = Mesh(np.array(devs[:ndev]), ("d",))
    f = shard_map(
        lambda pa, ra, pj, rj, mk, pk: _run(pa, ra, pj, rj, mk, pk,
                                            interpret, B // ndev),
        mesh=mesh,
        in_specs=(P("d"), P("d"), P("d"), P("d"), P("d"), P()),
        out_specs=P("d"),
        check_vma=False,
    )
    return f(patoms, ratoms, padjs, radjs, masks, packed)


def kernel(ratoms, patoms, radjs, padjs, rcoords, pcoords, tscoords, masks, params):
    return _forward(ratoms, patoms, radjs, padjs, masks, params)


# speedup vs baseline: 2.5457x; 2.5457x over previous
import functools

import jax
import jax.numpy as jnp
import numpy as np
from jax.experimental import pallas as pl
from jax.experimental.pallas import tpu as pltpu

AS, BT, HS, HV, ES, EV, NH, CB = 16, 5, 128, 16, 64, 8, 8, 3
B, N = 8, 128
DH = HS // NH
NE = N * N

_SELU_A = 1.6732632423543772
_SELU_S = 1.0507009873554805


def _selu(x):
    return _SELU_S * jnp.where(x > 0, x, _SELU_A * (jnp.exp(x) - 1.0))


def _ln_rows(z, s, b):
    mu = jnp.mean(z, axis=-1, keepdims=True)
    d = z - mu
    var = jnp.mean(d * d, axis=-1, keepdims=True)
    return d * jax.lax.rsqrt(var + 1e-5) * s + b


def _dot(a, b):
    return jnp.dot(a, b, preferred_element_type=jnp.float32)


def _fwd_kernel(
    patoms_ref, ratoms_ref, padjt_ref, radjt_ref, maskr_ref, maskc_ref,
    xembW_ref, xembB_ref, eembWT_ref, eembBc_ref,
    Wqkv_ref, Wog_ref, b128_ref, WedgeT_ref, WeT_ref, b64c_ref,
    gsxW_ref, gsxB_ref, gseWT_ref, gseBc_ref,
    tsW_ref, tsB_ref, ggW_ref, ggB_ref, ggWf_ref, ggBf_ref,
    out_ref, eP, eR,
):
    mask_row = maskr_ref[0]

    def layer_step(l, x, e_scr):
        E = e_scr[:, :]
        EB = _dot(WeT_ref[l], E)
        qkv = _dot(x, Wqkv_ref[l])
        qks = []
        for h in range(NH):
            q_h = qkv[:, h * DH:(h + 1) * DH]
            k_h = qkv[:, HS + h * DH:HS + (h + 1) * DH]
            qks.append(jax.lax.dot_general(
                q_h, k_h, (((1,), (1,)), ((), ())),
                preferred_element_type=jnp.float32))
        attns = []
        for h in range(NH):
            eb_h = EB[h:h + 1, :].reshape(1, N, N)[0]
            logits = qks[h] * (1.0 / np.sqrt(DH)) + eb_h
            logits = jnp.where(mask_row > 0, logits, -1e6)
            m = jnp.max(logits, axis=-1, keepdims=True)
            p = jnp.exp(logits - m)
            s = jnp.sum(p, axis=-1, keepdims=True)
            attns.append(p / s)
        outs = [_dot(attns[h], qkv[:, 2 * HS + h * DH:2 * HS + (h + 1) * DH])
                for h in range(NH)]
        out = jnp.concatenate(outs, axis=-1)
        out = _dot(out, Wog_ref[l, 0]) + b128_ref[l, 0]
        g = jax.nn.sigmoid(_dot(x, Wog_ref[l, 1]) + b128_ref[l, 1])
        x = _ln_rows(x + g * out, b128_ref[l, 2], b128_ref[l, 3])
        Hm = _selu(_dot(WedgeT_ref[l], E) + b64c_ref[l, 0])
        Z = E + Hm
        ones_row = jnp.full((1, ES), 1.0 / ES, dtype=jnp.float32)
        mu = _dot(ones_row, Z)
        ex2 = _dot(ones_row, Z * Z)
        var = ex2 - mu * mu
        e_scr[:, :] = ((Z - mu) * jax.lax.rsqrt(var + 1e-5) * b64c_ref[l, 1]
                       + b64c_ref[l, 2])
        return x

    def stack3(layer_base, x, e_scr):
        return jax.lax.fori_loop(
            layer_base, layer_base + CB,
            lambda l, xc: layer_step(l, xc, e_scr), x)

    def encode(enc_i, atoms_ref, adjt_ref, e_scr, layer_base):
        x = _dot(atoms_ref[0], xembW_ref[enc_i]) + xembB_ref[enc_i]
        e_scr[:, :] = _dot(eembWT_ref[enc_i], adjt_ref[0]) + eembBc_ref[enc_i]
        return stack3(layer_base, x, e_scr)

    p_x = encode(0, patoms_ref, padjt_ref, eP, 0)
    r_x = encode(1, ratoms_ref, radjt_ref, eR, CB)

    gx = jax.nn.sigmoid(_dot(r_x, gsxW_ref[0]) + _dot(p_x, gsxW_ref[1])
                        + gsxB_ref[0])
    xs = gx * r_x + (1.0 - gx) * p_x
    Er = eR[:, :]
    Ep = eP[:, :]
    ge = jax.nn.sigmoid(_dot(gseWT_ref[0], Er) + _dot(gseWT_ref[1], Ep)
                        + gseBc_ref[0])
    eR[:, :] = ge * Er + (1.0 - ge) * Ep

    xs = stack3(2 * CB, xs, eR)

    h = xs
    for i in range(3):
        h = _selu(_dot(h, tsW_ref[i]) + tsB_ref[i])

    a = _selu(_dot(h, ggW_ref[0]) + _dot(xs, ggW_ref[1]) + ggB_ref[0])
    for i in range(3):
        a = _selu(_dot(a, ggW_ref[2 + i]) + ggB_ref[1 + i])
    energies = _selu(_dot(a, ggWf_ref[0]) + ggBf_ref[0:1, 0:1])
    energies = energies - (maskc_ref[0] == 0).astype(jnp.float32) * 1e6
    m = jnp.max(energies, axis=0, keepdims=True)
    p = jnp.exp(energies - m)
    att = p / jnp.sum(p, axis=0, keepdims=True)

    b = _selu(_dot(h, ggW_ref[5]) + ggB_ref[4])
    for i in range(3):
        b = _selu(_dot(b, ggW_ref[6 + i]) + ggB_ref[5 + i])
    emb = _selu(_dot(b, ggWf_ref[1]) + ggBf_ref[1:2, 0:1])

    out_ref[0] = jnp.sum(att * emb, axis=0, keepdims=True)


def _pack(params):
    p_enc = params["rencoder"]
    r_enc = params["pencoder"]
    layers = list(p_enc["layers"]) + list(r_enc["layers"]) + list(params["ts_layers"])
    Wqkv = jnp.stack([
        jnp.concatenate([lp["Wq"], lp["Wk"], lp["Wv"]], axis=1)
        for lp in layers])
    Wog = jnp.stack([
        jnp.stack([lp["Wo"]["W"], lp["gate"]["W"]]) for lp in layers])
    b128 = jnp.stack([
        jnp.stack([lp["Wo"]["b"], lp["gate"]["b"], lp["ln_s"], lp["ln_b"]])
        for lp in layers])
    WedgeT = jnp.stack([lp["edge"]["W"].T for lp in layers])
    WeT = jnp.stack([lp["We"].T for lp in layers])
    b64c = jnp.stack([
        jnp.stack([lp["edge"]["b"], lp["lne_s"], lp["lne_b"]])
        for lp in layers])[..., None]
    xembW = jnp.stack([p_enc["x_emb"]["W"], r_enc["x_emb"]["W"]])
    xembB = jnp.stack([p_enc["x_emb"]["b"], r_enc["x_emb"]["b"]])
    eembWT = jnp.stack([p_enc["e_emb"]["W"].T, r_enc["e_emb"]["W"].T])
    eembBc = jnp.stack([p_enc["e_emb"]["b"], r_enc["e_emb"]["b"]])[..., None]
    gsx = params["x_gate"]["gs"]
    gsxW = jnp.stack([gsx["W"][:HS], gsx["W"][HS:]])
    gsxB = gsx["b"][None, :]
    gse = params["e_gate"]["gs"]
    gseWT = jnp.stack([gse["W"][:ES].T, gse["W"][ES:].T])
    gseBc = gse["b"][None, :, None]
    tsW = jnp.stack([lp["W"] for lp in params["tsmlp"]])
    tsB = jnp.stack([lp["b"] for lp in params["tsmlp"]])
    att = params["gg"]["att"]
    emb = params["gg"]["emb"]
    ggW = jnp.stack([att[0]["W"][:HS], att[0]["W"][HS:],
                     att[1]["W"], att[2]["W"], att[3]["W"],
                     emb[0]["W"], emb[1]["W"], emb[2]["W"], emb[3]["W"]])
    ggB = jnp.stack([att[0]["b"], att[1]["b"], att[2]["b"], att[3]["b"],
                     emb[0]["b"], emb[1]["b"], emb[2]["b"], emb[3]["b"]])
    ggWf = jnp.stack([att[4]["W"], emb[4]["W"]])
    ggBf = jnp.stack([att[4]["b"], emb[4]["b"]])
    return (xembW, xembB, eembWT, eembBc, Wqkv, Wog, b128, WedgeT, WeT, b64c,
            gsxW, gsxB, gseWT, gseBc, tsW, tsB, ggW, ggB, ggWf, ggBf)


def _full(shape):
    nd = len(shape)
    return pl.BlockSpec(shape, lambda b, _n=nd: (0,) * _n)


def _run(patoms, ratoms, padjs, radjs, masks, packed, interpret, bl):
    padjt = padjs.transpose(0, 3, 1, 2).reshape(bl, BT, NE)
    radjt = radjs.transpose(0, 3, 1, 2).reshape(bl, BT, NE)
    maskr = masks[:, None, :]
    maskc = masks[..., None]
    in_specs = [
        pl.BlockSpec((1, N, AS), lambda b: (b, 0, 0)),
        pl.BlockSpec((1, N, AS), lambda b: (b, 0, 0)),
        pl.BlockSpec((1, BT, NE), lambda b: (b, 0, 0)),
        pl.BlockSpec((1, BT, NE), lambda b: (b, 0, 0)),
        pl.BlockSpec((1, 1, N), lambda b: (b, 0, 0)),
        pl.BlockSpec((1, N, 1), lambda b: (b, 0, 0)),
    ] + [_full(w.shape) for w in packed]
    out = pl.pallas_call(
        _fwd_kernel,
        grid=(bl,),
        in_specs=in_specs,
        out_specs=pl.BlockSpec((1, 1, 1), lambda b: (b, 0, 0)),
        out_shape=jax.ShapeDtypeStruct((bl, 1, 1), jnp.float32),
        scratch_shapes=[pltpu.VMEM((ES, NE), jnp.float32),
                        pltpu.VMEM((ES, NE), jnp.float32)],
        compiler_params=pltpu.CompilerParams(
            dimension_semantics=("parallel",),
            vmem_limit_bytes=100 * 1024 * 1024,
        ),
        interpret=interpret,
    )(patoms, ratoms, padjt, radjt, maskr, maskc, *packed)
    return out.reshape(bl, 1)


@functools.partial(jax.jit, static_argnames=("interpret",))
def _forward(ratoms, patoms, radjs, padjs, masks, params, interpret=False):
    packed = _pack(params)
    return _run(patoms, ratoms, padjs, radjs, masks, packed, interpret, B)


def kernel(ratoms, patoms, radjs, padjs, rcoords, pcoords, tscoords, masks, params):
    return _forward(ratoms, patoms, radjs, padjs, masks, params)


# speedup vs baseline: 2.5752x; 1.0116x over previous
import functools

import jax
import jax.numpy as jnp
import numpy as np
from jax.experimental import pallas as pl
from jax.experimental.pallas import tpu as pltpu

AS, BT, HS, HV, ES, EV, NH, CB = 16, 5, 128, 16, 64, 8, 8, 3
B, N = 8, 128
DH = HS // NH
NE = N * N

_SELU_A = 1.6732632423543772
_SELU_S = 1.0507009873554805


def _selu(x):
    return _SELU_S * jnp.where(x > 0, x, _SELU_A * (jnp.exp(x) - 1.0))


def _ln_rows(z, s, b):
    mu = jnp.mean(z, axis=-1, keepdims=True)
    d = z - mu
    var = jnp.mean(d * d, axis=-1, keepdims=True)
    return d * jax.lax.rsqrt(var + 1e-5) * s + b


def _dot(a, b):
    return jnp.dot(a, b, preferred_element_type=jnp.float32)


def _fwd_kernel(
    patoms_ref, ratoms_ref, padjt_ref, radjt_ref, maskr_ref, maskc_ref,
    xembW_ref, xembB_ref, eembWT_ref, eembBc_ref,
    Wqkv_ref, Wog_ref, b128_ref, WedgeT_ref, WeT_ref, b64c_ref,
    gsxW_ref, gsxB_ref, gseWT_ref, gseBc_ref,
    tsW_ref, tsB_ref, ggW_ref, ggB_ref, ggWf_ref, ggBf_ref,
    out_ref, eP, eR,
):
    mask_row = maskr_ref[0]

    def layer_step(l, x, e_scr):
        E = e_scr[:, :]
        EB = _dot(WeT_ref[l], E)
        qkv = _dot(x, Wqkv_ref[l])
        qks = []
        for h in range(NH):
            q_h = qkv[:, h * DH:(h + 1) * DH]
            k_h = qkv[:, HS + h * DH:HS + (h + 1) * DH]
            qks.append(jax.lax.dot_general(
                q_h, k_h, (((1,), (1,)), ((), ())),
                preferred_element_type=jnp.float32))
        attns = []
        for h in range(NH):
            eb_h = EB[h:h + 1, :].reshape(1, N, N)[0]
            logits = qks[h] * (1.0 / np.sqrt(DH)) + eb_h
            logits = jnp.where(mask_row > 0, logits, -1e6)
            m = jnp.max(logits, axis=-1, keepdims=True)
            p = jnp.exp(logits - m)
            s = jnp.sum(p, axis=-1, keepdims=True)
            attns.append(p / s)
        outs = [_dot(attns[h], qkv[:, 2 * HS + h * DH:2 * HS + (h + 1) * DH])
                for h in range(NH)]
        out = jnp.concatenate(outs, axis=-1)
        out = _dot(out, Wog_ref[l, 0]) + b128_ref[l, 0]
        g = jax.nn.sigmoid(_dot(x, Wog_ref[l, 1]) + b128_ref[l, 1])
        x = _ln_rows(x + g * out, b128_ref[l, 2], b128_ref[l, 3])
        Hm = _selu(_dot(WedgeT_ref[l], E) + b64c_ref[l, 0])
        Z = E + Hm
        ones_row = jnp.full((1, ES), 1.0 / ES, dtype=jnp.float32)
        mu = _dot(ones_row, Z)
        ex2 = _dot(ones_row, Z * Z)
        var = ex2 - mu * mu
        e_scr[:, :] = ((Z - mu) * jax.lax.rsqrt(var + 1e-5) * b64c_ref[l, 1]
                       + b64c_ref[l, 2])
        return x

    def stack3(layer_base, x, e_scr):
        return jax.lax.fori_loop(
            layer_base, layer_base + CB,
            lambda l, xc: layer_step(l, xc, e_scr), x)

    p_x = _dot(patoms_ref[0], xembW_ref[0]) + xembB_ref[0]
    eP[:, :] = _dot(eembWT_ref[0], padjt_ref[0]) + eembBc_ref[0]
    r_x = _dot(ratoms_ref[0], xembW_ref[1]) + xembB_ref[1]
    eR[:, :] = _dot(eembWT_ref[1], radjt_ref[0]) + eembBc_ref[1]

    def pair_body(l, carry):
        xp, xr = carry
        xp = layer_step(l, xp, eP)
        xr = layer_step(CB + l, xr, eR)
        return (xp, xr)

    p_x, r_x = jax.lax.fori_loop(0, CB, pair_body, (p_x, r_x))

    gx = jax.nn.sigmoid(_dot(r_x, gsxW_ref[0]) + _dot(p_x, gsxW_ref[1])
                        + gsxB_ref[0])
    xs = gx * r_x + (1.0 - gx) * p_x
    Er = eR[:, :]
    Ep = eP[:, :]
    ge = jax.nn.sigmoid(_dot(gseWT_ref[0], Er) + _dot(gseWT_ref[1], Ep)
                        + gseBc_ref[0])
    eR[:, :] = ge * Er + (1.0 - ge) * Ep

    xs = stack3(2 * CB, xs, eR)

    h = xs
    for i in range(3):
        h = _selu(_dot(h, tsW_ref[i]) + tsB_ref[i])

    a = _selu(_dot(h, ggW_ref[0]) + _dot(xs, ggW_ref[1]) + ggB_ref[0])
    for i in range(3):
        a = _selu(_dot(a, ggW_ref[2 + i]) + ggB_ref[1 + i])
    energies = _selu(_dot(a, ggWf_ref[0]) + ggBf_ref[0:1, 0:1])
    energies = energies - (maskc_ref[0] == 0).astype(jnp.float32) * 1e6
    m = jnp.max(energies, axis=0, keepdims=True)
    p = jnp.exp(energies - m)
    att = p / jnp.sum(p, axis=0, keepdims=True)

    b = _selu(_dot(h, ggW_ref[5]) + ggB_ref[4])
    for i in range(3):
        b = _selu(_dot(b, ggW_ref[6 + i]) + ggB_ref[5 + i])
    emb = _selu(_dot(b, ggWf_ref[1]) + ggBf_ref[1:2, 0:1])

    out_ref[0] = jnp.sum(att * emb, axis=0, keepdims=True)


def _pack(params):
    p_enc = params["rencoder"]
    r_enc = params["pencoder"]
    layers = list(p_enc["layers"]) + list(r_enc["layers"]) + list(params["ts_layers"])
    Wqkv = jnp.stack([
        jnp.concatenate([lp["Wq"], lp["Wk"], lp["Wv"]], axis=1)
        for lp in layers])
    Wog = jnp.stack([
        jnp.stack([lp["Wo"]["W"], lp["gate"]["W"]]) for lp in layers])
    b128 = jnp.stack([
        jnp.stack([lp["Wo"]["b"], lp["gate"]["b"], lp["ln_s"], lp["ln_b"]])
        for lp in layers])
    WedgeT = jnp.stack([lp["edge"]["W"].T for lp in layers])
    WeT = jnp.stack([lp["We"].T for lp in layers])
    b64c = jnp.stack([
        jnp.stack([lp["edge"]["b"], lp["lne_s"], lp["lne_b"]])
        for lp in layers])[..., None]
    xembW = jnp.stack([p_enc["x_emb"]["W"], r_enc["x_emb"]["W"]])
    xembB = jnp.stack([p_enc["x_emb"]["b"], r_enc["x_emb"]["b"]])
    eembWT = jnp.stack([p_enc["e_emb"]["W"].T, r_enc["e_emb"]["W"].T])
    eembBc = jnp.stack([p_enc["e_emb"]["b"], r_enc["e_emb"]["b"]])[..., None]
    gsx = params["x_gate"]["gs"]
    gsxW = jnp.stack([gsx["W"][:HS], gsx["W"][HS:]])
    gsxB = gsx["b"][None, :]
    gse = params["e_gate"]["gs"]
    gseWT = jnp.stack([gse["W"][:ES].T, gse["W"][ES:].T])
    gseBc = gse["b"][None, :, None]
    tsW = jnp.stack([lp["W"] for lp in params["tsmlp"]])
    tsB = jnp.stack([lp["b"] for lp in params["tsmlp"]])
    att = params["gg"]["att"]
    emb = params["gg"]["emb"]
    ggW = jnp.stack([att[0]["W"][:HS], att[0]["W"][HS:],
                     att[1]["W"], att[2]["W"], att[3]["W"],
                     emb[0]["W"], emb[1]["W"], emb[2]["W"], emb[3]["W"]])
    ggB = jnp.stack([att[0]["b"], att[1]["b"], att[2]["b"], att[3]["b"],
                     emb[0]["b"], emb[1]["b"], emb[2]["b"], emb[3]["b"]])
    ggWf = jnp.stack([att[4]["W"], emb[4]["W"]])
    ggBf = jnp.stack([att[4]["b"], emb[4]["b"]])
    return (xembW, xembB, eembWT, eembBc, Wqkv, Wog, b128, WedgeT, WeT, b64c,
            gsxW, gsxB, gseWT, gseBc, tsW, tsB, ggW, ggB, ggWf, ggBf)


def _full(shape):
    nd = len(shape)
    return pl.BlockSpec(shape, lambda b, _n=nd: (0,) * _n)


def _run(patoms, ratoms, padjs, radjs, masks, packed, interpret, bl):
    padjt = padjs.transpose(0, 3, 1, 2).reshape(bl, BT, NE)
    radjt = radjs.transpose(0, 3, 1, 2).reshape(bl, BT, NE)
    maskr = masks[:, None, :]
    maskc = masks[..., None]
    in_specs = [
        pl.BlockSpec((1, N, AS), lambda b: (b, 0, 0)),
        pl.BlockSpec((1, N, AS), lambda b: (b, 0, 0)),
        pl.BlockSpec((1, BT, NE), lambda b: (b, 0, 0)),
        pl.BlockSpec((1, BT, NE), lambda b: (b, 0, 0)),
        pl.BlockSpec((1, 1, N), lambda b: (b, 0, 0)),
        pl.BlockSpec((1, N, 1), lambda b: (b, 0, 0)),
    ] + [_full(w.shape) for w in packed]
    out = pl.pallas_call(
        _fwd_kernel,
        grid=(bl,),
        in_specs=in_specs,
        out_specs=pl.BlockSpec((1, 1, 1), lambda b: (b, 0, 0)),
        out_shape=jax.ShapeDtypeStruct((bl, 1, 1), jnp.float32),
        scratch_shapes=[pltpu.VMEM((ES, NE), jnp.float32),
                        pltpu.VMEM((ES, NE), jnp.float32)],
        compiler_params=pltpu.CompilerParams(
            dimension_semantics=("parallel",),
            vmem_limit_bytes=100 * 1024 * 1024,
        ),
        interpret=interpret,
    )(patoms, ratoms, padjt, radjt, maskr, maskc, *packed)
    return out.reshape(bl, 1)


@functools.partial(jax.jit, static_argnames=("interpret",))
def _forward(ratoms, patoms, radjs, padjs, masks, params, interpret=False):
    packed = _pack(params)
    return _run(patoms, ratoms, padjs, radjs, masks, packed, interpret, B)


def kernel(ratoms, patoms, radjs, padjs, rcoords, pcoords, tscoords, masks, params):
    return _forward(ratoms, patoms, radjs, padjs, masks, params)


# speedup vs baseline: 2.6098x; 1.0134x over previous
import functools

import jax
import jax.numpy as jnp
import numpy as np
from jax.experimental import pallas as pl
from jax.experimental.pallas import tpu as pltpu

AS, BT, HS, HV, ES, EV, NH, CB = 16, 5, 128, 16, 64, 8, 8, 3
B, N = 8, 128
DH = HS // NH
NE = N * N

_SELU_A = 1.6732632423543772
_SELU_S = 1.0507009873554805


def _selu(x):
    return _SELU_S * jnp.where(x > 0, x, _SELU_A * (jnp.exp(x) - 1.0))


def _ln_rows(z, s, b):
    mu = jnp.mean(z, axis=-1, keepdims=True)
    d = z - mu
    var = jnp.mean(d * d, axis=-1, keepdims=True)
    return d * jax.lax.rsqrt(var + 1e-5) * s + b


def _dot(a, b):
    return jnp.dot(a, b, preferred_element_type=jnp.float32)


def _fwd_kernel(
    patoms_ref, ratoms_ref, padjt_ref, radjt_ref, maskr_ref, maskc_ref,
    xembW_ref, xembB_ref, eembWT_ref, eembBc_ref,
    Wqkv_ref, Wog_ref, b128_ref, WedgeT_ref, WeT_ref, b64c_ref,
    gsxW_ref, gsxB_ref, gseWT_ref, gseBc_ref,
    tsW_ref, tsB_ref, ggW_ref, ggB_ref, ggWf_ref, ggBf_ref,
    out_ref, eP0, eP1, eR0, eR1,
):
    ePs, eRs = (eP0, eP1), (eR0, eR1)
    mrows = (maskr_ref[0], maskr_ref[1])

    def layer_step(l, x, e_scr, mask_row):
        E = e_scr[:, :]
        EB = _dot(WeT_ref[l], E)
        qkv = _dot(x, Wqkv_ref[l])
        qks = []
        for h in range(NH):
            q_h = qkv[:, h * DH:(h + 1) * DH]
            k_h = qkv[:, HS + h * DH:HS + (h + 1) * DH]
            qks.append(jax.lax.dot_general(
                q_h, k_h, (((1,), (1,)), ((), ())),
                preferred_element_type=jnp.float32))
        attns = []
        for h in range(NH):
            eb_h = EB[h:h + 1, :].reshape(1, N, N)[0]
            logits = qks[h] * (1.0 / np.sqrt(DH)) + eb_h
            logits = jnp.where(mask_row > 0, logits, -1e6)
            m = jnp.max(logits, axis=-1, keepdims=True)
            p = jnp.exp(logits - m)
            s = jnp.sum(p, axis=-1, keepdims=True)
            attns.append(p / s)
        outs = [_dot(attns[h], qkv[:, 2 * HS + h * DH:2 * HS + (h + 1) * DH])
                for h in range(NH)]
        out = jnp.concatenate(outs, axis=-1)
        out = _dot(out, Wog_ref[l, 0]) + b128_ref[l, 0]
        g = jax.nn.sigmoid(_dot(x, Wog_ref[l, 1]) + b128_ref[l, 1])
        x = _ln_rows(x + g * out, b128_ref[l, 2], b128_ref[l, 3])
        Hm = _selu(_dot(WedgeT_ref[l], E) + b64c_ref[l, 0])
        Z = E + Hm
        ones_row = jnp.full((1, ES), 1.0 / ES, dtype=jnp.float32)
        mu = _dot(ones_row, Z)
        ex2 = _dot(ones_row, Z * Z)
        var = ex2 - mu * mu
        e_scr[:, :] = ((Z - mu) * jax.lax.rsqrt(var + 1e-5) * b64c_ref[l, 1]
                       + b64c_ref[l, 2])
        return x

    xps, xrs = [], []
    for m in range(2):
        xps.append(_dot(patoms_ref[m], xembW_ref[0]) + xembB_ref[0])
        ePs[m][:, :] = _dot(eembWT_ref[0], padjt_ref[m]) + eembBc_ref[0]
        xrs.append(_dot(ratoms_ref[m], xembW_ref[1]) + xembB_ref[1])
        eRs[m][:, :] = _dot(eembWT_ref[1], radjt_ref[m]) + eembBc_ref[1]

    def enc_body(l, carry):
        xp0, xp1, xr0, xr1 = carry
        xp0 = layer_step(l, xp0, eP0, mrows[0])
        xp1 = layer_step(l, xp1, eP1, mrows[1])
        xr0 = layer_step(CB + l, xr0, eR0, mrows[0])
        xr1 = layer_step(CB + l, xr1, eR1, mrows[1])
        return (xp0, xp1, xr0, xr1)

    xp0, xp1, xr0, xr1 = jax.lax.fori_loop(
        0, CB, enc_body, (xps[0], xps[1], xrs[0], xrs[1]))

    xss = []
    for m, (xp, xr) in enumerate(((xp0, xr0), (xp1, xr1))):
        gx = jax.nn.sigmoid(_dot(xr, gsxW_ref[0]) + _dot(xp, gsxW_ref[1])
                            + gsxB_ref[0])
        xss.append(gx * xr + (1.0 - gx) * xp)
        Er = eRs[m][:, :]
        Ep = ePs[m][:, :]
        ge = jax.nn.sigmoid(_dot(gseWT_ref[0], Er) + _dot(gseWT_ref[1], Ep)
                            + gseBc_ref[0])
        eRs[m][:, :] = ge * Er + (1.0 - ge) * Ep

    def ts_body(l, carry):
        x0, x1 = carry
        x0 = layer_step(2 * CB + l, x0, eR0, mrows[0])
        x1 = layer_step(2 * CB + l, x1, eR1, mrows[1])
        return (x0, x1)

    xs0, xs1 = jax.lax.fori_loop(0, CB, ts_body, (xss[0], xss[1]))

    for m, xs in enumerate((xs0, xs1)):
        h = xs
        for i in range(3):
            h = _selu(_dot(h, tsW_ref[i]) + tsB_ref[i])

        a = _selu(_dot(h, ggW_ref[0]) + _dot(xs, ggW_ref[1]) + ggB_ref[0])
        for i in range(3):
            a = _selu(_dot(a, ggW_ref[2 + i]) + ggB_ref[1 + i])
        energies = _selu(_dot(a, ggWf_ref[0]) + ggBf_ref[0:1, 0:1])
        energies = energies - (maskc_ref[m] == 0).astype(jnp.float32) * 1e6
        mx = jnp.max(energies, axis=0, keepdims=True)
        p = jnp.exp(energies - mx)
        att = p / jnp.sum(p, axis=0, keepdims=True)

        b = _selu(_dot(h, ggW_ref[5]) + ggB_ref[4])
        for i in range(3):
            b = _selu(_dot(b, ggW_ref[6 + i]) + ggB_ref[5 + i])
        emb = _selu(_dot(b, ggWf_ref[1]) + ggBf_ref[1:2, 0:1])

        out_ref[m] = jnp.sum(att * emb, axis=0, keepdims=True)


def _pack(params):
    p_enc = params["rencoder"]
    r_enc = params["pencoder"]
    layers = list(p_enc["layers"]) + list(r_enc["layers"]) + list(params["ts_layers"])
    Wqkv = jnp.stack([
        jnp.concatenate([lp["Wq"], lp["Wk"], lp["Wv"]], axis=1)
        for lp in layers])
    Wog = jnp.stack([
        jnp.stack([lp["Wo"]["W"], lp["gate"]["W"]]) for lp in layers])
    b128 = jnp.stack([
        jnp.stack([lp["Wo"]["b"], lp["gate"]["b"], lp["ln_s"], lp["ln_b"]])
        for lp in layers])
    WedgeT = jnp.stack([lp["edge"]["W"].T for lp in layers])
    WeT = jnp.stack([lp["We"].T for lp in layers])
    b64c = jnp.stack([
        jnp.stack([lp["edge"]["b"], lp["lne_s"], lp["lne_b"]])
        for lp in layers])[..., None]
    xembW = jnp.stack([p_enc["x_emb"]["W"], r_enc["x_emb"]["W"]])
    xembB = jnp.stack([p_enc["x_emb"]["b"], r_enc["x_emb"]["b"]])
    eembWT = jnp.stack([p_enc["e_emb"]["W"].T, r_enc["e_emb"]["W"].T])
    eembBc = jnp.stack([p_enc["e_emb"]["b"], r_enc["e_emb"]["b"]])[..., None]
    gsx = params["x_gate"]["gs"]
    gsxW = jnp.stack([gsx["W"][:HS], gsx["W"][HS:]])
    gsxB = gsx["b"][None, :]
    gse = params["e_gate"]["gs"]
    gseWT = jnp.stack([gse["W"][:ES].T, gse["W"][ES:].T])
    gseBc = gse["b"][None, :, None]
    tsW = jnp.stack([lp["W"] for lp in params["tsmlp"]])
    tsB = jnp.stack([lp["b"] for lp in params["tsmlp"]])
    att = params["gg"]["att"]
    emb = params["gg"]["emb"]
    ggW = jnp.stack([att[0]["W"][:HS], att[0]["W"][HS:],
                     att[1]["W"], att[2]["W"], att[3]["W"],
                     emb[0]["W"], emb[1]["W"], emb[2]["W"], emb[3]["W"]])
    ggB = jnp.stack([att[0]["b"], att[1]["b"], att[2]["b"], att[3]["b"],
                     emb[0]["b"], emb[1]["b"], emb[2]["b"], emb[3]["b"]])
    ggWf = jnp.stack([att[4]["W"], emb[4]["W"]])
    ggBf = jnp.stack([att[4]["b"], emb[4]["b"]])
    return (xembW, xembB, eembWT, eembBc, Wqkv, Wog, b128, WedgeT, WeT, b64c,
            gsxW, gsxB, gseWT, gseBc, tsW, tsB, ggW, ggB, ggWf, ggBf)


def _full(shape):
    nd = len(shape)
    return pl.BlockSpec(shape, lambda b, _n=nd: (0,) * _n)


def _run(patoms, ratoms, padjs, radjs, masks, packed, interpret, bl):
    padjt = padjs.transpose(0, 3, 1, 2).reshape(bl, BT, NE)
    radjt = radjs.transpose(0, 3, 1, 2).reshape(bl, BT, NE)
    maskr = masks[:, None, :]
    maskc = masks[..., None]
    in_specs = [
        pl.BlockSpec((2, N, AS), lambda b: (b, 0, 0)),
        pl.BlockSpec((2, N, AS), lambda b: (b, 0, 0)),
        pl.BlockSpec((2, BT, NE), lambda b: (b, 0, 0)),
        pl.BlockSpec((2, BT, NE), lambda b: (b, 0, 0)),
        pl.BlockSpec((2, 1, N), lambda b: (b, 0, 0)),
        pl.BlockSpec((2, N, 1), lambda b: (b, 0, 0)),
    ] + [_full(w.shape) for w in packed]
    out = pl.pallas_call(
        _fwd_kernel,
        grid=(bl // 2,),
        in_specs=in_specs,
        out_specs=pl.BlockSpec((2, 1, 1), lambda b: (b, 0, 0)),
        out_shape=jax.ShapeDtypeStruct((bl, 1, 1), jnp.float32),
        scratch_shapes=[pltpu.VMEM((ES, NE), jnp.float32)] * 4,
        compiler_params=pltpu.CompilerParams(
            dimension_semantics=("parallel",),
            vmem_limit_bytes=100 * 1024 * 1024,
        ),
        interpret=interpret,
    )(patoms, ratoms, padjt, radjt, maskr, maskc, *packed)
    return out.reshape(bl, 1)


@functools.partial(jax.jit, static_argnames=("interpret",))
def _forward(ratoms, patoms, radjs, padjs, masks, params, interpret=False):
    packed = _pack(params)
    return _run(patoms, ratoms, padjs, radjs, masks, packed, interpret, B)


def kernel(ratoms, patoms, radjs, padjs, rcoords, pcoords, tscoords, masks, params):
    return _forward(ratoms, patoms, radjs, padjs, masks, params)
